# baseline (speedup 1.0000x reference)
import functools

import jax
import jax.numpy as jnp
from jax import lax
from jax.experimental import pallas as pl
from jax.experimental.pallas import tpu as pltpu
from jax.experimental.pallas import tpu_sc as plsc

_THRESHOLD = 0.5

_NC = 2
_NS = 16
_NW = _NC * _NS

_ROWS = 2 * 192 * 384
_W = 384
_ROWS_PER_W = _ROWS // _NW
_CH = 128
_N_CH = _ROWS_PER_W // _CH


def _sc_copy(x_hbm, o_hbm, buf0, buf1, si0, si1, so0, so1):
    wid = lax.axis_index("s") * _NC + lax.axis_index("c")
    base = wid * _ROWS_PER_W
    bufs = (buf0, buf1)
    in_sems = (si0, si1)
    out_sems = (so0, so1)

    def start_in(i):
        return pltpu.async_copy(
            x_hbm.at[pl.ds(base + i * _CH, _CH)], bufs[i % 2], in_sems[i % 2])

    def start_out(i):
        return pltpu.async_copy(
            bufs[i % 2], o_hbm.at[pl.ds(base + i * _CH, _CH)], out_sems[i % 2])

    out_copies = [None, None]
    in_copy = start_in(0)
    for i in range(_N_CH):
        b = i % 2
        nb = (i + 1) % 2
        cur_in = in_copy
        if i + 1 < _N_CH:
            if out_copies[nb] is not None:
                out_copies[nb].wait()
            in_copy = start_in(i + 1)
        cur_in.wait()
        out_copies[b] = start_out(i)
    for c in out_copies:
        if c is not None:
            c.wait()


def kernel(x, heatmap0):
    del heatmap0
    b, c, h, w = x.shape
    x2 = x.reshape(_ROWS, _W)
    mesh = plsc.VectorSubcoreMesh(core_axis_name="c", subcore_axis_name="s")
    run = functools.partial(
        pl.kernel,
        out_type=jax.ShapeDtypeStruct((_ROWS, _W), x.dtype),
        mesh=mesh,
        scratch_types=[
            pltpu.VMEM((_CH, _W), jnp.float32),
            pltpu.VMEM((_CH, _W), jnp.float32),
            pltpu.SemaphoreType.DMA,
            pltpu.SemaphoreType.DMA,
            pltpu.SemaphoreType.DMA,
            pltpu.SemaphoreType.DMA,
        ],
    )(_sc_copy)
    out = run(x2)
    return (out.reshape(b, c, h, w), jnp.float32(_THRESHOLD))
